# baseline (speedup 1.0000x reference)
import functools

import jax
import jax.numpy as jnp
from jax import lax
from jax.experimental import pallas as pl
from jax.experimental.pallas import tpu as pltpu
from jax.experimental.pallas import tpu_sc as plsc

N = 10000
E = 320000
F_IN = 128
H = 256
G = 64
HH = H // 2
NC = 2
NT = 16
NPAD = 10112
STRIPE = NPAD // NT
PAD_ROW = N

CHUNK = 96
BCH = 24
NB = 9
CPT = BCH * NB
EPT = CPT * CHUNK
DCPT = 105
DEPT = DCPT * CHUNK

def _sc_mesh():
    return plsc.VectorSubcoreMesh(core_axis_name="c", subcore_axis_name="s")


def _deg_body(dst_hbm, ones_hbm, zeros_hbm, out_hbm, dst_v, ones_v, acc, sem):
    cid = lax.axis_index("c")
    tid = lax.axis_index("s")
    g = cid * NT + tid
    pltpu.sync_copy(dst_hbm.at[g], dst_v)
    pltpu.sync_copy(ones_hbm, ones_v)
    pltpu.sync_copy(zeros_hbm.at[pl.ds(tid * STRIPE, STRIPE)],
                    acc.at[pl.ds(tid * STRIPE, STRIPE)])
    plsc.subcore_barrier()

    def body(c, carry):
        pltpu.sync_copy(ones_v, acc.at[dst_v.at[c]], add=True)
        return carry

    lax.fori_loop(0, DCPT, body, 0)
    plsc.subcore_barrier()
    pltpu.sync_copy(acc.at[pl.ds(tid * STRIPE, STRIPE)],
                    out_hbm.at[cid, pl.ds(tid * STRIPE, STRIPE)])


def _sc_deg(dst3, ones128, zerosH):
    kern = pl.kernel(
        _deg_body,
        mesh=_sc_mesh(),
        out_type=jax.ShapeDtypeStruct((NC, NPAD, HH), jnp.float32),
        scratch_types=[
            pltpu.VMEM((DCPT, CHUNK), jnp.int32),
            pltpu.VMEM((CHUNK, HH), jnp.float32),
            pltpu.VMEM_SHARED((NPAD, HH), jnp.float32),
            pltpu.SemaphoreType.DMA,
        ],
    )
    return kern(dst3, ones128, zerosH)


def _agg_body(table_hbm, src_hbm, dst_hbm, zeros_hbm, out_hbm,
              src_v, dst_v, buf0, buf1, buf2, acc,
              gsem0, gsem1, gsem2, ssem0, ssem1, ssem2):
    cid = lax.axis_index("c")
    tid = lax.axis_index("s")
    bufs = (buf0, buf1, buf2)
    gsems = (gsem0, gsem1, gsem2)
    ssems = (ssem0, ssem1, ssem2)
    pltpu.sync_copy(zeros_hbm.at[pl.ds(tid * STRIPE, STRIPE)],
                    acc.at[pl.ds(tid * STRIPE, STRIPE)])
    plsc.subcore_barrier()

    def batch_body(b, carry):
        pltpu.sync_copy(src_hbm.at[cid, tid, pl.ds(b * BCH, BCH)], src_v)
        pltpu.sync_copy(dst_hbm.at[tid, pl.ds(b * BCH, BCH)], dst_v)
        pltpu.async_copy(table_hbm.at[src_v.at[0]], buf0, gsem0)
        pltpu.async_copy(table_hbm.at[src_v.at[1]], buf1, gsem1)

        def tri_body(t, carry2):
            for j in range(3):
                c = 3 * t + j
                jn = (j + 2) % 3
                pltpu.make_async_copy(table_hbm.at[src_v.at[c]],
                                      bufs[j], gsems[j]).wait()
                pltpu.async_copy(bufs[j], acc.at[dst_v.at[c]], ssems[j],
                                 add=True)

                @pl.when(c + 2 < BCH)
                def _():
                    @pl.when(c > 0)
                    def _():
                        pltpu.make_async_copy(bufs[jn], acc.at[dst_v.at[c]],
                                              ssems[jn]).wait()
                    pltpu.async_copy(table_hbm.at[src_v.at[c + 2]],
                                     bufs[jn], gsems[jn])
            return carry2

        lax.fori_loop(0, BCH // 3, tri_body, carry)
        for j in range(3):
            pltpu.make_async_copy(bufs[j], acc.at[dst_v.at[0]],
                                  ssems[j]).wait()
        return carry

    lax.fori_loop(0, NB, batch_body, 0)
    plsc.subcore_barrier()
    pltpu.sync_copy(acc.at[pl.ds(tid * STRIPE, STRIPE)],
                    out_hbm.at[cid, pl.ds(tid * STRIPE, STRIPE)])


def _sc_agg(table, src3, dst3, zerosH):
    kern = pl.kernel(
        _agg_body,
        mesh=_sc_mesh(),
        out_type=jax.ShapeDtypeStruct((NC, NPAD, HH), jnp.float32),
        scratch_types=[
            pltpu.VMEM((BCH, CHUNK), jnp.int32),
            pltpu.VMEM((BCH, CHUNK), jnp.int32),
            pltpu.VMEM((CHUNK, HH), jnp.float32),
            pltpu.VMEM((CHUNK, HH), jnp.float32),
            pltpu.VMEM((CHUNK, HH), jnp.float32),
            pltpu.VMEM_SHARED((NPAD, HH), jnp.float32),
            pltpu.SemaphoreType.DMA,
            pltpu.SemaphoreType.DMA,
            pltpu.SemaphoreType.DMA,
            pltpu.SemaphoreType.DMA,
            pltpu.SemaphoreType.DMA,
            pltpu.SemaphoreType.DMA,
        ],
    )
    return kern(table, src3, dst3, zerosH)


TM = 1000


def _mm_bf16(a, w):
    return jnp.dot(a.astype(jnp.bfloat16), w.astype(jnp.bfloat16),
                   preferred_element_type=jnp.float32)


def _enc_body(x_ref, w_ref, b_ref, o_ref):
    y = _mm_bf16(x_ref[...], w_ref[...])
    o_ref[...] = jnp.maximum(y + b_ref[...], 0.0)


def _k_enc(x, w, b):
    return pl.pallas_call(
        _enc_body,
        grid=(N // TM,),
        in_specs=[
            pl.BlockSpec((TM, F_IN), lambda i: (i, 0)),
            pl.BlockSpec((F_IN, H), lambda i: (0, 0)),
            pl.BlockSpec((1, H), lambda i: (0, 0)),
        ],
        out_specs=pl.BlockSpec((TM, H), lambda i: (i, 0)),
        out_shape=jax.ShapeDtypeStruct((N, H), jnp.float32),
    )(x, w, b.reshape(1, H))


def _dinv_of(deg_blk):
    d = 1.0 + deg_blk[0, :, :1] + deg_blk[1, :, :1]
    r = lax.rsqrt(d)
    return r * (1.5 - 0.5 * d * r * r)


def _mm_scale_body(h_ref, w_ref, deg_ref, o_ref):
    y = _mm_bf16(h_ref[...], w_ref[...])
    y = y * _dinv_of(deg_ref[...])
    o_ref[0] = y[:, :HH]
    o_ref[1] = y[:, HH:]


def _k_mm_scale(h, w, deg):
    return pl.pallas_call(
        _mm_scale_body,
        grid=(N // TM,),
        in_specs=[
            pl.BlockSpec((TM, H), lambda i: (i, 0)),
            pl.BlockSpec((H, H), lambda i: (0, 0)),
            pl.BlockSpec((NC, TM, HH), lambda i: (0, i, 0)),
        ],
        out_specs=pl.BlockSpec((NC, TM, HH), lambda i: (0, i, 0)),
        out_shape=jax.ShapeDtypeStruct((NC, N, HH), jnp.float32),
    )(h, w, deg)


def _combine_body(agg_ref, hp_ref, deg_ref, b_ref, o_ref):
    dinv = _dinv_of(deg_ref[...])
    s0 = (agg_ref[0] + hp_ref[0]) * dinv
    s1 = (agg_ref[1] + hp_ref[1]) * dinv
    y = jnp.concatenate([s0, s1], axis=1) + b_ref[...]
    o_ref[...] = jnp.maximum(y, 0.0)


def _k_combine(agg, hp, deg, b):
    return pl.pallas_call(
        _combine_body,
        grid=(N // TM,),
        in_specs=[
            pl.BlockSpec((NC, TM, HH), lambda i: (0, i, 0)),
            pl.BlockSpec((NC, TM, HH), lambda i: (0, i, 0)),
            pl.BlockSpec((NC, TM, HH), lambda i: (0, i, 0)),
            pl.BlockSpec((1, H), lambda i: (0, 0)),
        ],
        out_specs=pl.BlockSpec((TM, H), lambda i: (i, 0)),
        out_shape=jax.ShapeDtypeStruct((N, H), jnp.float32),
    )(agg, hp, deg, b.reshape(1, H))


def _head_body(h_ref, batch_ref, w1_ref, b1_ref, w2_ref, b2_ref, o_ref):
    gids = lax.broadcasted_iota(jnp.int32, (G, 1), 0).astype(jnp.float32)
    ind = (gids == batch_ref[...]).astype(jnp.float32)
    summed = jnp.dot(ind, h_ref[...],
                     preferred_element_type=jnp.float32,
                     precision=lax.Precision.HIGHEST)
    counts = jnp.sum(ind, axis=1, keepdims=True)
    pooled = summed / jnp.maximum(counts, 1.0)
    m = jnp.maximum(_mm_bf16(pooled, w1_ref[...]) + b1_ref[...], 0.0)
    o_ref[...] = _mm_bf16(m, w2_ref[...]) + b2_ref[...]


def _k_head(h, batch_f, w1, b1, w2p, b2p):
    return pl.pallas_call(
        _head_body,
        grid=(1,),
        in_specs=[
            pl.BlockSpec((N, H), lambda i: (0, 0)),
            pl.BlockSpec((1, N), lambda i: (0, 0)),
            pl.BlockSpec((H, H), lambda i: (0, 0)),
            pl.BlockSpec((1, H), lambda i: (0, 0)),
            pl.BlockSpec((H, 128), lambda i: (0, 0)),
            pl.BlockSpec((1, 128), lambda i: (0, 0)),
        ],
        out_specs=pl.BlockSpec((G, 128), lambda i: (0, 0)),
        out_shape=jax.ShapeDtypeStruct((G, 128), jnp.float32),
    )(h, batch_f, w1, b1.reshape(1, H), w2p, b2p.reshape(1, 128))


def kernel(x, edge_index, batch, W_enc, b_enc, W_g1, b_g1, W_g2, b_g2,
           W_g3, b_g3, W_m1, b_m1, W_m2, b_m2):
    src = edge_index[0]
    dst = edge_index[1]

    pad_a = NT * EPT - E
    src_p = jnp.concatenate([src, jnp.zeros((pad_a,), jnp.int32)])
    dst_p = jnp.concatenate([dst, jnp.full((pad_a,), PAD_ROW, jnp.int32)])
    src3 = jnp.stack([src_p, src_p + N]).reshape(NC, NT, CPT, CHUNK)
    dst3 = dst_p.reshape(NT, CPT, CHUNK)

    pad_d = NC * NT * DEPT - E
    dstd = jnp.concatenate([dst, jnp.full((pad_d,), PAD_ROW, jnp.int32)])
    dstd = dstd.reshape(NC * NT, DCPT, CHUNK)

    ones128 = jnp.ones((CHUNK, HH), jnp.float32)
    zerosH = jnp.zeros((NPAD, HH), jnp.float32)

    deg = _sc_deg(dstd, ones128, zerosH)
    h = _k_enc(x, W_enc, b_enc)

    for (W, b) in ((W_g1, b_g1), (W_g2, b_g2), (W_g3, b_g3)):
        hp = _k_mm_scale(h, W, deg[:, :N, :])
        agg = _sc_agg(hp.reshape(NC * N, HH), src3, dst3, zerosH)
        h = _k_combine(agg[:, :N, :], hp, deg[:, :N, :], b)

    batch_f = batch.astype(jnp.float32).reshape(1, N)
    w2p = jnp.pad(W_m2, ((0, 0), (0, 127)))
    b2p = jnp.pad(b_m2, (0, 127))
    out = _k_head(h, batch_f, W_m1, b_m1, w2p, b2p)
    return out[:, 0]


# speedup vs baseline: 1.2300x; 1.2300x over previous
import functools

import jax
import jax.numpy as jnp
from jax import lax
from jax.experimental import pallas as pl
from jax.experimental.pallas import tpu as pltpu
from jax.experimental.pallas import tpu_sc as plsc

N = 10000
E = 320000
F_IN = 128
H = 256
G = 64
HH = H // 2
NC = 2
NT = 16
NPAD = 10112
STRIPE = NPAD // NT
PAD_ROW = N

CHUNK = 128
BCH = 32
NB = 5
CPT = BCH * NB
EPT = CPT * CHUNK
DCPT = 79
DEPT = DCPT * CHUNK

def _sc_mesh():
    return plsc.VectorSubcoreMesh(core_axis_name="c", subcore_axis_name="s")


def _deg_body(dst_hbm, ones_hbm, zeros_hbm, out_hbm, dst_v, ones_v, acc, sem):
    cid = lax.axis_index("c")
    tid = lax.axis_index("s")
    g = cid * NT + tid
    pltpu.sync_copy(dst_hbm.at[g], dst_v)
    pltpu.sync_copy(ones_hbm, ones_v)
    pltpu.sync_copy(zeros_hbm.at[pl.ds(tid * STRIPE, STRIPE)],
                    acc.at[pl.ds(tid * STRIPE, STRIPE)])
    plsc.subcore_barrier()

    def body(c, carry):
        pltpu.sync_copy(ones_v, acc.at[dst_v.at[c]], add=True)
        return carry

    lax.fori_loop(0, DCPT, body, 0)
    plsc.subcore_barrier()
    pltpu.sync_copy(acc.at[pl.ds(tid * STRIPE, STRIPE)],
                    out_hbm.at[cid, pl.ds(tid * STRIPE, STRIPE)])


def _sc_deg(dst3, ones128, zerosH):
    kern = pl.kernel(
        _deg_body,
        mesh=_sc_mesh(),
        out_type=jax.ShapeDtypeStruct((NC, NPAD, HH), jnp.float32),
        scratch_types=[
            pltpu.VMEM((DCPT, CHUNK), jnp.int32),
            pltpu.VMEM((CHUNK, HH), jnp.float32),
            pltpu.VMEM_SHARED((NPAD, HH), jnp.float32),
            pltpu.SemaphoreType.DMA,
        ],
    )
    return kern(dst3, ones128, zerosH)


def _agg_body(table_hbm, src_hbm, dst_hbm, zeros_hbm, out_hbm,
              src_v, dst_v, buf0, buf1, acc, sem0, sem1):
    cid = lax.axis_index("c")
    tid = lax.axis_index("s")
    bufs = (buf0, buf1)
    sems = (sem0, sem1)
    pltpu.sync_copy(zeros_hbm.at[pl.ds(tid * STRIPE, STRIPE)],
                    acc.at[pl.ds(tid * STRIPE, STRIPE)])
    plsc.subcore_barrier()

    def batch_body(b, carry):
        pltpu.sync_copy(src_hbm.at[cid, tid, pl.ds(b * BCH, BCH)], src_v)
        pltpu.sync_copy(dst_hbm.at[tid, pl.ds(b * BCH, BCH)], dst_v)
        pltpu.async_copy(table_hbm.at[src_v.at[0]], buf0, sem0)
        pltpu.async_copy(table_hbm.at[src_v.at[1]], buf1, sem1)

        def pair_body(p, carry2):
            for j in range(2):
                c = 2 * p + j
                pltpu.make_async_copy(table_hbm.at[src_v.at[c]],
                                      bufs[j], sems[j]).wait()
                pltpu.sync_copy(bufs[j], acc.at[dst_v.at[c]], add=True)

                @pl.when(c + 2 < BCH)
                def _():
                    pltpu.async_copy(table_hbm.at[src_v.at[c + 2]],
                                     bufs[j], sems[j])
            return carry2

        return lax.fori_loop(0, BCH // 2, pair_body, carry)

    lax.fori_loop(0, NB, batch_body, 0)
    plsc.subcore_barrier()
    pltpu.sync_copy(acc.at[pl.ds(tid * STRIPE, STRIPE)],
                    out_hbm.at[cid, pl.ds(tid * STRIPE, STRIPE)])


def _sc_agg(table, src3, dst3, zerosH):
    kern = pl.kernel(
        _agg_body,
        mesh=_sc_mesh(),
        out_type=jax.ShapeDtypeStruct((NC, NPAD, HH), jnp.float32),
        scratch_types=[
            pltpu.VMEM((BCH, CHUNK), jnp.int32),
            pltpu.VMEM((BCH, CHUNK), jnp.int32),
            pltpu.VMEM((CHUNK, HH), jnp.float32),
            pltpu.VMEM((CHUNK, HH), jnp.float32),
            pltpu.VMEM_SHARED((NPAD, HH), jnp.float32),
            pltpu.SemaphoreType.DMA,
            pltpu.SemaphoreType.DMA,
        ],
    )
    return kern(table, src3, dst3, zerosH)


TM = 1000


def _mm_bf16(a, w):
    return jnp.dot(a.astype(jnp.bfloat16), w.astype(jnp.bfloat16),
                   preferred_element_type=jnp.float32)


def _enc_body(x_ref, w_ref, b_ref, o_ref):
    y = _mm_bf16(x_ref[...], w_ref[...])
    o_ref[...] = jnp.maximum(y + b_ref[...], 0.0)


def _k_enc(x, w, b):
    return pl.pallas_call(
        _enc_body,
        grid=(N // TM,),
        in_specs=[
            pl.BlockSpec((TM, F_IN), lambda i: (i, 0)),
            pl.BlockSpec((F_IN, H), lambda i: (0, 0)),
            pl.BlockSpec((1, H), lambda i: (0, 0)),
        ],
        out_specs=pl.BlockSpec((TM, H), lambda i: (i, 0)),
        out_shape=jax.ShapeDtypeStruct((N, H), jnp.float32),
    )(x, w, b.reshape(1, H))


def _dinv_of(deg_blk):
    d = 1.0 + deg_blk[0, :, :1] + deg_blk[1, :, :1]
    r = lax.rsqrt(d)
    return r * (1.5 - 0.5 * d * r * r)


def _mm_scale_body(h_ref, w_ref, deg_ref, o_ref):
    y = _mm_bf16(h_ref[...], w_ref[...])
    y = y * _dinv_of(deg_ref[...])
    o_ref[0] = y[:, :HH]
    o_ref[1] = y[:, HH:]


def _k_mm_scale(h, w, deg):
    return pl.pallas_call(
        _mm_scale_body,
        grid=(N // TM,),
        in_specs=[
            pl.BlockSpec((TM, H), lambda i: (i, 0)),
            pl.BlockSpec((H, H), lambda i: (0, 0)),
            pl.BlockSpec((NC, TM, HH), lambda i: (0, i, 0)),
        ],
        out_specs=pl.BlockSpec((NC, TM, HH), lambda i: (0, i, 0)),
        out_shape=jax.ShapeDtypeStruct((NC, N, HH), jnp.float32),
    )(h, w, deg)


def _mm_combine_scale_body(agg_ref, hp_ref, deg_ref, b_ref, w_ref, o_ref):
    dinv = _dinv_of(deg_ref[...])
    s0 = (agg_ref[0] + hp_ref[0]) * dinv
    s1 = (agg_ref[1] + hp_ref[1]) * dinv
    h = jnp.maximum(jnp.concatenate([s0, s1], axis=1) + b_ref[...], 0.0)
    y = _mm_bf16(h, w_ref[...]) * dinv
    o_ref[0] = y[:, :HH]
    o_ref[1] = y[:, HH:]


def _k_mm_combine_scale(agg, hp, deg, b, w):
    return pl.pallas_call(
        _mm_combine_scale_body,
        grid=(N // TM,),
        in_specs=[
            pl.BlockSpec((NC, TM, HH), lambda i: (0, i, 0)),
            pl.BlockSpec((NC, TM, HH), lambda i: (0, i, 0)),
            pl.BlockSpec((NC, TM, HH), lambda i: (0, i, 0)),
            pl.BlockSpec((1, H), lambda i: (0, 0)),
            pl.BlockSpec((H, H), lambda i: (0, 0)),
        ],
        out_specs=pl.BlockSpec((NC, TM, HH), lambda i: (0, i, 0)),
        out_shape=jax.ShapeDtypeStruct((NC, N, HH), jnp.float32),
    )(agg, hp, deg, b.reshape(1, H), w)


def _combine_body(agg_ref, hp_ref, deg_ref, b_ref, o_ref):
    dinv = _dinv_of(deg_ref[...])
    s0 = (agg_ref[0] + hp_ref[0]) * dinv
    s1 = (agg_ref[1] + hp_ref[1]) * dinv
    y = jnp.concatenate([s0, s1], axis=1) + b_ref[...]
    o_ref[...] = jnp.maximum(y, 0.0)


def _k_combine(agg, hp, deg, b):
    return pl.pallas_call(
        _combine_body,
        grid=(N // TM,),
        in_specs=[
            pl.BlockSpec((NC, TM, HH), lambda i: (0, i, 0)),
            pl.BlockSpec((NC, TM, HH), lambda i: (0, i, 0)),
            pl.BlockSpec((NC, TM, HH), lambda i: (0, i, 0)),
            pl.BlockSpec((1, H), lambda i: (0, 0)),
        ],
        out_specs=pl.BlockSpec((TM, H), lambda i: (i, 0)),
        out_shape=jax.ShapeDtypeStruct((N, H), jnp.float32),
    )(agg, hp, deg, b.reshape(1, H))


def _head_body(h_ref, batch_ref, w1_ref, b1_ref, w2_ref, b2_ref, o_ref):
    gids = lax.broadcasted_iota(jnp.int32, (G, 1), 0).astype(jnp.float32)
    ind = (gids == batch_ref[...]).astype(jnp.float32)
    summed = jnp.dot(ind, h_ref[...],
                     preferred_element_type=jnp.float32,
                     precision=lax.Precision.HIGHEST)
    counts = jnp.sum(ind, axis=1, keepdims=True)
    pooled = summed / jnp.maximum(counts, 1.0)
    m = jnp.maximum(_mm_bf16(pooled, w1_ref[...]) + b1_ref[...], 0.0)
    o_ref[...] = _mm_bf16(m, w2_ref[...]) + b2_ref[...]


def _k_head(h, batch_f, w1, b1, w2p, b2p):
    return pl.pallas_call(
        _head_body,
        grid=(1,),
        in_specs=[
            pl.BlockSpec((N, H), lambda i: (0, 0)),
            pl.BlockSpec((1, N), lambda i: (0, 0)),
            pl.BlockSpec((H, H), lambda i: (0, 0)),
            pl.BlockSpec((1, H), lambda i: (0, 0)),
            pl.BlockSpec((H, 128), lambda i: (0, 0)),
            pl.BlockSpec((1, 128), lambda i: (0, 0)),
        ],
        out_specs=pl.BlockSpec((G, 128), lambda i: (0, 0)),
        out_shape=jax.ShapeDtypeStruct((G, 128), jnp.float32),
    )(h, batch_f, w1, b1.reshape(1, H), w2p, b2p.reshape(1, 128))


def kernel(x, edge_index, batch, W_enc, b_enc, W_g1, b_g1, W_g2, b_g2,
           W_g3, b_g3, W_m1, b_m1, W_m2, b_m2):
    src = edge_index[0]
    dst = edge_index[1]

    pad_a = NT * EPT - E
    src_p = jnp.concatenate([src, jnp.zeros((pad_a,), jnp.int32)])
    dst_p = jnp.concatenate([dst, jnp.full((pad_a,), PAD_ROW, jnp.int32)])
    src3 = jnp.stack([src_p, src_p + N]).reshape(NC, NT, CPT, CHUNK)
    dst3 = dst_p.reshape(NT, CPT, CHUNK)

    pad_d = NC * NT * DEPT - E
    dstd = jnp.concatenate([dst, jnp.full((pad_d,), PAD_ROW, jnp.int32)])
    dstd = dstd.reshape(NC * NT, DCPT, CHUNK)

    ones128 = jnp.ones((CHUNK, HH), jnp.float32)
    zerosH = jnp.zeros((NPAD, HH), jnp.float32)

    deg = _sc_deg(dstd, ones128, zerosH)
    degN = deg[:, :N, :]
    h = _k_enc(x, W_enc, b_enc)

    hp = _k_mm_scale(h, W_g1, degN)
    agg = _sc_agg(hp.reshape(NC * N, HH), src3, dst3, zerosH)
    for (W, b_prev) in ((W_g2, b_g1), (W_g3, b_g2)):
        hp = _k_mm_combine_scale(agg[:, :N, :], hp, degN, b_prev, W)
        agg = _sc_agg(hp.reshape(NC * N, HH), src3, dst3, zerosH)
    h = _k_combine(agg[:, :N, :], hp, degN, b_g3)

    batch_f = batch.astype(jnp.float32).reshape(1, N)
    w2p = jnp.pad(W_m2, ((0, 0), (0, 127)))
    b2p = jnp.pad(b_m2, (0, 127))
    out = _k_head(h, batch_f, W_m1, b_m1, w2p, b2p)
    return out[:, 0]


# speedup vs baseline: 1.2367x; 1.0055x over previous
import functools

import jax
import jax.numpy as jnp
from jax import lax
from jax.experimental import pallas as pl
from jax.experimental.pallas import tpu as pltpu
from jax.experimental.pallas import tpu_sc as plsc

N = 10000
E = 320000
F_IN = 128
H = 256
G = 64
HH = H // 2
NC = 2
NT = 16
NPAD = 10112
STRIPE = NPAD // NT
PAD_ROW = N

CHUNK = 128
BCH = 32
NB = 5
CPT = BCH * NB
EPT = CPT * CHUNK
DCPT = 79
DEPT = DCPT * CHUNK

def _sc_mesh():
    return plsc.VectorSubcoreMesh(core_axis_name="c", subcore_axis_name="s")


def _deg_body(dst_hbm, ones_hbm, zeros_hbm, out_hbm, dst_v, ones_v, acc, sem):
    cid = lax.axis_index("c")
    tid = lax.axis_index("s")
    g = cid * NT + tid
    pltpu.sync_copy(dst_hbm.at[g], dst_v)
    pltpu.sync_copy(ones_hbm, ones_v)
    pltpu.sync_copy(zeros_hbm.at[pl.ds(tid * STRIPE, STRIPE)],
                    acc.at[pl.ds(tid * STRIPE, STRIPE)])
    plsc.subcore_barrier()

    def body(c, carry):
        pltpu.sync_copy(ones_v, acc.at[dst_v.at[c]], add=True)
        return carry

    lax.fori_loop(0, DCPT, body, 0)
    plsc.subcore_barrier()
    pltpu.sync_copy(acc.at[pl.ds(tid * STRIPE, STRIPE)],
                    out_hbm.at[cid, pl.ds(tid * STRIPE, STRIPE)])


def _sc_deg(dst3, ones128, zerosH):
    kern = pl.kernel(
        _deg_body,
        mesh=_sc_mesh(),
        out_type=jax.ShapeDtypeStruct((NC, NPAD, HH), jnp.float32),
        scratch_types=[
            pltpu.VMEM((DCPT, CHUNK), jnp.int32),
            pltpu.VMEM((CHUNK, HH), jnp.float32),
            pltpu.VMEM_SHARED((NPAD, HH), jnp.float32),
            pltpu.SemaphoreType.DMA,
        ],
    )
    return kern(dst3, ones128, zerosH)


def _agg_body(table_hbm, src_hbm, dst_hbm, zeros_hbm, out_hbm,
              src_v, dst_v, buf0, buf1, acc, sem0, sem1):
    cid = lax.axis_index("c")
    tid = lax.axis_index("s")
    bufs = (buf0, buf1)
    sems = (sem0, sem1)
    pltpu.sync_copy(zeros_hbm.at[pl.ds(tid * STRIPE, STRIPE)],
                    acc.at[pl.ds(tid * STRIPE, STRIPE)])
    plsc.subcore_barrier()

    def batch_body(b, carry):
        pltpu.sync_copy(src_hbm.at[cid, tid, pl.ds(b * BCH, BCH)], src_v)
        pltpu.sync_copy(dst_hbm.at[tid, pl.ds(b * BCH, BCH)], dst_v)
        pltpu.async_copy(table_hbm.at[src_v.at[0]], buf0, sem0)
        pltpu.async_copy(table_hbm.at[src_v.at[1]], buf1, sem1)

        def pair_body(p, carry2):
            for j in range(2):
                c = 2 * p + j
                pltpu.make_async_copy(table_hbm.at[src_v.at[c]],
                                      bufs[j], sems[j]).wait()
                pltpu.sync_copy(bufs[j], acc.at[dst_v.at[c]], add=True)

                @pl.when(c + 2 < BCH)
                def _():
                    pltpu.async_copy(table_hbm.at[src_v.at[c + 2]],
                                     bufs[j], sems[j])
            return carry2

        return lax.fori_loop(0, BCH // 2, pair_body, carry)

    lax.fori_loop(0, NB, batch_body, 0)
    plsc.subcore_barrier()
    pltpu.sync_copy(acc.at[pl.ds(tid * STRIPE, STRIPE)],
                    out_hbm.at[cid, pl.ds(tid * STRIPE, STRIPE)])


def _sc_agg(table, src3, dst3, zerosH):
    kern = pl.kernel(
        _agg_body,
        mesh=_sc_mesh(),
        out_type=jax.ShapeDtypeStruct((NC, NPAD, HH), jnp.float32),
        scratch_types=[
            pltpu.VMEM((BCH, CHUNK), jnp.int32),
            pltpu.VMEM((BCH, CHUNK), jnp.int32),
            pltpu.VMEM((CHUNK, HH), jnp.float32),
            pltpu.VMEM((CHUNK, HH), jnp.float32),
            pltpu.VMEM_SHARED((NPAD, HH), jnp.float32),
            pltpu.SemaphoreType.DMA,
            pltpu.SemaphoreType.DMA,
        ],
    )
    return kern(table, src3, dst3, zerosH)


TM = 1000


def _mm_bf16(a, w):
    return jnp.dot(a.astype(jnp.bfloat16), w.astype(jnp.bfloat16),
                   preferred_element_type=jnp.float32)


def _enc_body(x_ref, w_ref, b_ref, o_ref):
    y = _mm_bf16(x_ref[...], w_ref[...])
    o_ref[...] = jnp.maximum(y + b_ref[...], 0.0)


def _k_enc(x, w, b):
    return pl.pallas_call(
        _enc_body,
        grid=(N // TM,),
        in_specs=[
            pl.BlockSpec((TM, F_IN), lambda i: (i, 0)),
            pl.BlockSpec((F_IN, H), lambda i: (0, 0)),
            pl.BlockSpec((1, H), lambda i: (0, 0)),
        ],
        out_specs=pl.BlockSpec((TM, H), lambda i: (i, 0)),
        out_shape=jax.ShapeDtypeStruct((N, H), jnp.float32),
    )(x, w, b.reshape(1, H))


def _dinv_of(deg_blk):
    d = 1.0 + deg_blk[0, :, :1] + deg_blk[1, :, :1]
    r = lax.rsqrt(d)
    return r * (1.5 - 0.5 * d * r * r)


def _mm_scale_body(h_ref, w_ref, deg_ref, o_ref):
    y = _mm_bf16(h_ref[...], w_ref[...])
    y = y * _dinv_of(deg_ref[...])
    o_ref[0] = y[:, :HH]
    o_ref[1] = y[:, HH:]


def _k_mm_scale(h, w, deg):
    return pl.pallas_call(
        _mm_scale_body,
        grid=(N // TM,),
        in_specs=[
            pl.BlockSpec((TM, H), lambda i: (i, 0)),
            pl.BlockSpec((H, H), lambda i: (0, 0)),
            pl.BlockSpec((NC, TM, HH), lambda i: (0, i, 0)),
        ],
        out_specs=pl.BlockSpec((NC, TM, HH), lambda i: (0, i, 0)),
        out_shape=jax.ShapeDtypeStruct((NC, N, HH), jnp.float32),
    )(h, w, deg)


def _mm_combine_scale_body(agg_ref, hp_ref, deg_ref, b_ref, w_ref, o_ref):
    dinv = _dinv_of(deg_ref[...])
    s0 = (agg_ref[0] + hp_ref[0]) * dinv
    s1 = (agg_ref[1] + hp_ref[1]) * dinv
    h = jnp.maximum(jnp.concatenate([s0, s1], axis=1) + b_ref[...], 0.0)
    y = _mm_bf16(h, w_ref[...]) * dinv
    o_ref[0] = y[:, :HH]
    o_ref[1] = y[:, HH:]


def _k_mm_combine_scale(agg, hp, deg, b, w):
    return pl.pallas_call(
        _mm_combine_scale_body,
        grid=(N // TM,),
        in_specs=[
            pl.BlockSpec((NC, TM, HH), lambda i: (0, i, 0)),
            pl.BlockSpec((NC, TM, HH), lambda i: (0, i, 0)),
            pl.BlockSpec((NC, TM, HH), lambda i: (0, i, 0)),
            pl.BlockSpec((1, H), lambda i: (0, 0)),
            pl.BlockSpec((H, H), lambda i: (0, 0)),
        ],
        out_specs=pl.BlockSpec((NC, TM, HH), lambda i: (0, i, 0)),
        out_shape=jax.ShapeDtypeStruct((NC, N, HH), jnp.float32),
    )(agg, hp, deg, b.reshape(1, H), w)


def _combine_body(agg_ref, hp_ref, deg_ref, b_ref, o_ref):
    dinv = _dinv_of(deg_ref[...])
    s0 = (agg_ref[0] + hp_ref[0]) * dinv
    s1 = (agg_ref[1] + hp_ref[1]) * dinv
    y = jnp.concatenate([s0, s1], axis=1) + b_ref[...]
    o_ref[...] = jnp.maximum(y, 0.0)


def _k_combine(agg, hp, deg, b):
    return pl.pallas_call(
        _combine_body,
        grid=(N // TM,),
        in_specs=[
            pl.BlockSpec((NC, TM, HH), lambda i: (0, i, 0)),
            pl.BlockSpec((NC, TM, HH), lambda i: (0, i, 0)),
            pl.BlockSpec((NC, TM, HH), lambda i: (0, i, 0)),
            pl.BlockSpec((1, H), lambda i: (0, 0)),
        ],
        out_specs=pl.BlockSpec((TM, H), lambda i: (i, 0)),
        out_shape=jax.ShapeDtypeStruct((N, H), jnp.float32),
    )(agg, hp, deg, b.reshape(1, H))


def _head_body(agg_ref, hp_ref, deg_ref, b_ref, batch_ref,
               w1_ref, b1_ref, w2_ref, b2_ref, o_ref):
    dinv = _dinv_of(deg_ref[...])
    s0 = (agg_ref[0] + hp_ref[0]) * dinv
    s1 = (agg_ref[1] + hp_ref[1]) * dinv
    h = jnp.maximum(jnp.concatenate([s0, s1], axis=1) + b_ref[...], 0.0)
    gids = lax.broadcasted_iota(jnp.int32, (G, 1), 0).astype(jnp.float32)
    ind = (gids == batch_ref[...]).astype(jnp.float32)
    summed = jnp.dot(ind, h,
                     preferred_element_type=jnp.float32,
                     precision=lax.Precision.HIGHEST)
    counts = jnp.sum(ind, axis=1, keepdims=True)
    pooled = summed / jnp.maximum(counts, 1.0)
    m = jnp.maximum(_mm_bf16(pooled, w1_ref[...]) + b1_ref[...], 0.0)
    o_ref[...] = _mm_bf16(m, w2_ref[...]) + b2_ref[...]


def _k_head(agg, hp, deg, b, batch_f, w1, b1, w2p, b2p):
    return pl.pallas_call(
        _head_body,
        grid=(1,),
        in_specs=[
            pl.BlockSpec((NC, N, HH), lambda i: (0, 0, 0)),
            pl.BlockSpec((NC, N, HH), lambda i: (0, 0, 0)),
            pl.BlockSpec((NC, N, HH), lambda i: (0, 0, 0)),
            pl.BlockSpec((1, H), lambda i: (0, 0)),
            pl.BlockSpec((1, N), lambda i: (0, 0)),
            pl.BlockSpec((H, H), lambda i: (0, 0)),
            pl.BlockSpec((1, H), lambda i: (0, 0)),
            pl.BlockSpec((H, 128), lambda i: (0, 0)),
            pl.BlockSpec((1, 128), lambda i: (0, 0)),
        ],
        out_specs=pl.BlockSpec((G, 128), lambda i: (0, 0)),
        out_shape=jax.ShapeDtypeStruct((G, 128), jnp.float32),
    )(agg, hp, deg, b.reshape(1, H), batch_f, w1, b1.reshape(1, H),
      w2p, b2p.reshape(1, 128))


def kernel(x, edge_index, batch, W_enc, b_enc, W_g1, b_g1, W_g2, b_g2,
           W_g3, b_g3, W_m1, b_m1, W_m2, b_m2):
    src = edge_index[0]
    dst = edge_index[1]

    pad_a = NT * EPT - E
    src_p = jnp.concatenate([src, jnp.zeros((pad_a,), jnp.int32)])
    dst_p = jnp.concatenate([dst, jnp.full((pad_a,), PAD_ROW, jnp.int32)])
    src3 = jnp.stack([src_p, src_p + N]).reshape(NC, NT, CPT, CHUNK)
    dst3 = dst_p.reshape(NT, CPT, CHUNK)

    pad_d = NC * NT * DEPT - E
    dstd = jnp.concatenate([dst, jnp.full((pad_d,), PAD_ROW, jnp.int32)])
    dstd = dstd.reshape(NC * NT, DCPT, CHUNK)

    ones128 = jnp.ones((CHUNK, HH), jnp.float32)
    zerosH = jnp.zeros((NPAD, HH), jnp.float32)

    deg = _sc_deg(dstd, ones128, zerosH)
    degN = deg[:, :N, :]
    h = _k_enc(x, W_enc, b_enc)

    hp = _k_mm_scale(h, W_g1, degN)
    agg = _sc_agg(hp.reshape(NC * N, HH), src3, dst3, zerosH)
    for (W, b_prev) in ((W_g2, b_g1), (W_g3, b_g2)):
        hp = _k_mm_combine_scale(agg[:, :N, :], hp, degN, b_prev, W)
        agg = _sc_agg(hp.reshape(NC * N, HH), src3, dst3, zerosH)
    batch_f = batch.astype(jnp.float32).reshape(1, N)
    w2p = jnp.pad(W_m2, ((0, 0), (0, 127)))
    b2p = jnp.pad(b_m2, (0, 127))
    out = _k_head(agg[:, :N, :], hp, degN, b_g3, batch_f, W_m1, b_m1,
                  w2p, b2p)
    return out[:, 0]
